# Initial kernel scaffold; baseline (speedup 1.0000x reference)
import jax
import jax.numpy as jnp
from jax.experimental import pallas as pl


def kernel(traj, traj_rel, time_mask, is_predictable_mask, same_scene_mask, h0, c0, W_pos, b_pos, W_soc, b_soc, W_ih, W_hh, b_ih, b_hh, W_pred, b_pred):
    raise NotImplementedError("write your pallas kernel here")


# speedup vs baseline: 37.0329x; 37.0329x over previous
import jax
import jax.numpy as jnp
from jax.experimental import pallas as pl

_GRID = 2
_SPAN = 2.0
_EMBED = 64
_HIDDEN = 128
_PRED_LEN = 12
_N = 256
_OBS_LEN = 8

_PREC = jax.lax.Precision.HIGHEST


def _dot(a, b):
    return jax.lax.dot_general(a, b, (((1,), (0,)), ((), ())),
                               precision=_PREC, preferred_element_type=jnp.float32)


def _outer_row(ones_col, col):
    return jax.lax.dot_general(ones_col, col, (((1,), (1,)), ((), ())),
                               precision=_PREC, preferred_element_type=jnp.float32)


def _body(tx, ty, rx, ry, sidf, kmf, h0b, c0b, wposT, bpos,
          ws0, ws1, ws2, ws3, bsoc, wie, wia, whh, bg, wpred, bpredr, out_ref):
    N, H = _N, _HIDDEN
    ones_col = jnp.ones((N, 1), jnp.float32)

    sid_col = sidf[...]
    sid_row_full = _outer_row(ones_col, sid_col)
    same = sid_row_full == sid_col
    ri = jax.lax.broadcasted_iota(jnp.int32, (N, N), 0)
    ci = jax.lax.broadcasted_iota(jnp.int32, (N, N), 1)
    base_static = same & (ri != ci)

    wpos0 = wposT[0:1, :]
    wpos1 = wposT[1:2, :]
    bpos_r = bpos[...]
    bsoc_r = bsoc[...]
    bg_r = bg[...]
    wie_m = wie[...]
    wia_m = wia[...]
    whh_m = whh[...]
    ws = (ws0[...], ws1[...], ws2[...], ws3[...])

    lim = _SPAN / 2.0 - 0.01

    def social(pxc, pyc, ht):
        relx = _outer_row(ones_col, pxc) - pxc
        rely = _outer_row(ones_col, pyc) - pyc
        within = (relx < lim) & (relx > -lim) & (rely < lim) & (rely > -lim)
        base = within & base_static
        gx = jnp.floor(relx + _SPAN / 2.0)
        gy = jnp.floor(rely + _SPAN / 2.0)
        at_pre = bsoc_r
        for g in range(_GRID * _GRID):
            cell = (gx == float(g // _GRID)) & (gy == float(g % _GRID))
            mg = (base & cell).astype(jnp.float32)
            pooled_g = _dot(mg, ht)
            at_pre = at_pre + _dot(pooled_g, ws[g])
        return jax.nn.relu(at_pre)

    def lstm(et, at, ht, ct):
        g = _dot(et, wie_m) + _dot(at, wia_m) + _dot(ht, whh_m) + bg_r
        ii = jax.nn.sigmoid(g[:, 0 * H:1 * H])
        ff = jax.nn.sigmoid(g[:, 1 * H:2 * H])
        gg = jnp.tanh(g[:, 2 * H:3 * H])
        oo = jax.nn.sigmoid(g[:, 3 * H:4 * H])
        c_new = ff * ct + ii * gg
        h_new = oo * jnp.tanh(c_new)
        return h_new, c_new

    ht = ones_col * h0b[...]
    ct = ones_col * c0b[...]

    for t in range(_OBS_LEN):
        pxc = tx[:, t:t + 1]
        pyc = ty[:, t:t + 1]
        at = social(pxc, pyc, ht)
        et = jax.nn.relu(rx[:, t:t + 1] * wpos0 + ry[:, t:t + 1] * wpos1 + bpos_r)
        ht, ct = lstm(et, at, ht, ct)

    pxc = tx[:, _OBS_LEN - 1:_OBS_LEN]
    pyc = ty[:, _OBS_LEN - 1:_OBS_LEN]
    kmf_r = kmf[...]
    for t in range(_PRED_LEN):
        o = _dot(ht, wpred[...]) + bpredr[...]
        out_ref[t] = o * kmf_r
        dx = o[:, 0:1]
        dy = o[:, 1:2]
        pxc = pxc + dx
        pyc = pyc + dy
        at = social(pxc, pyc, ht)
        et = jax.nn.relu(dx * wpos0 + dy * wpos1 + bpos_r)
        ht, ct = lstm(et, at, ht, ct)


def _prep(traj, traj_rel, time_mask, is_predictable_mask, same_scene_mask, h0, c0,
          W_pos, b_pos, W_soc, b_soc, W_ih, W_hh, b_ih, b_hh, W_pred, b_pred):
    f32 = jnp.float32
    tx = traj[:, :, 0].astype(f32)
    ty = traj[:, :, 1].astype(f32)
    rx = traj_rel[:, :, 0].astype(f32)
    ry = traj_rel[:, :, 1].astype(f32)
    sidf = same_scene_mask.astype(f32)
    kmf = (is_predictable_mask == 1).astype(f32)
    h0b = h0.reshape(1, _HIDDEN)
    c0b = c0.reshape(1, _HIDDEN)
    wposT = W_pos.T
    bpos = b_pos.reshape(1, _EMBED)
    ws = [W_soc[:, g * _HIDDEN:(g + 1) * _HIDDEN].T for g in range(_GRID * _GRID)]
    bsoc = b_soc.reshape(1, _EMBED)
    wie = W_ih[:, :_EMBED].T
    wia = W_ih[:, _EMBED:].T
    whh = W_hh.T
    bg = (b_ih + b_hh).reshape(1, 4 * _HIDDEN)
    wpred = W_pred.T
    bpredr = b_pred.reshape(1, 5)
    return (tx, ty, rx, ry, sidf, kmf, h0b, c0b, wposT, bpos,
            ws[0], ws[1], ws[2], ws[3], bsoc, wie, wia, whh, bg, wpred, bpredr)


def kernel(traj, traj_rel, time_mask, is_predictable_mask, same_scene_mask, h0, c0,
           W_pos, b_pos, W_soc, b_soc, W_ih, W_hh, b_ih, b_hh, W_pred, b_pred):
    ops = _prep(traj, traj_rel, time_mask, is_predictable_mask, same_scene_mask,
                h0, c0, W_pos, b_pos, W_soc, b_soc, W_ih, W_hh, b_ih, b_hh,
                W_pred, b_pred)
    out = pl.pallas_call(
        _body,
        out_shape=jax.ShapeDtypeStruct((_PRED_LEN, _N, 5), jnp.float32),
    )(*ops)
    return jnp.transpose(out, (1, 0, 2))


# speedup vs baseline: 43.2882x; 1.1689x over previous
import jax
import jax.numpy as jnp
from jax.experimental import pallas as pl

_GRID = 2
_SPAN = 2.0
_EMBED = 64
_HIDDEN = 128
_PRED_LEN = 12
_N = 256
_OBS_LEN = 8

_PREC = jax.lax.Precision.HIGHEST


def _dot(a, b):
    return jax.lax.dot_general(a, b, (((1,), (0,)), ((), ())),
                               precision=_PREC, preferred_element_type=jnp.float32)


def _dot_fast(a, b):
    return jax.lax.dot_general(a, b, (((1,), (0,)), ((), ())),
                               precision=jax.lax.Precision.DEFAULT,
                               preferred_element_type=jnp.float32)


def _split3(x):
    x1 = x.astype(jnp.bfloat16)
    r1 = x - x1.astype(jnp.float32)
    x2 = r1.astype(jnp.bfloat16)
    x3 = (r1 - x2.astype(jnp.float32)).astype(jnp.bfloat16)
    return jnp.concatenate([x1, x2, x3], axis=1)


def _outer_row(ones_col, col):
    return jax.lax.dot_general(ones_col, col, (((1,), (1,)), ((), ())),
                               precision=_PREC, preferred_element_type=jnp.float32)


def _body(tx, ty, rx, ry, sidf, kmf, h0b, c0b, wposT, bpos,
          ws0, ws1, ws2, ws3, bsoc, wie, wia, whh, bg, wpred, bpredr, out_ref):
    N, H = _N, _HIDDEN
    ones_col = jnp.ones((N, 1), jnp.float32)

    sid_col = sidf[...]
    sid_row_full = _outer_row(ones_col, sid_col)
    same = sid_row_full == sid_col
    ri = jax.lax.broadcasted_iota(jnp.int32, (N, N), 0)
    ci = jax.lax.broadcasted_iota(jnp.int32, (N, N), 1)
    base_static = same & (ri != ci)

    wpos0 = wposT[0:1, :]
    wpos1 = wposT[1:2, :]
    bpos_r = bpos[...]
    bsoc_r = bsoc[...]
    bg_r = bg[...]
    wie_m = wie[...]
    wia_m = wia[...]
    whh_m = whh[...]
    ws = (ws0[...], ws1[...], ws2[...], ws3[...])

    lim = _SPAN / 2.0 - 0.01

    def social(pxc, pyc, ht):
        relx = _outer_row(ones_col, pxc) - pxc
        rely = _outer_row(ones_col, pyc) - pyc
        within = (relx < lim) & (relx > -lim) & (rely < lim) & (rely > -lim)
        base = within & base_static
        gx = jnp.floor(relx + _SPAN / 2.0)
        gy = jnp.floor(rely + _SPAN / 2.0)
        gid = gx * float(_GRID) + gy
        hs = _split3(ht)
        at_pre = bsoc_r
        for g in range(_GRID * _GRID):
            mg = (base & (gid == float(g))).astype(jnp.bfloat16)
            parts = _dot_fast(mg, hs)
            pooled_g = parts[:, 0:H] + parts[:, H:2 * H] + parts[:, 2 * H:3 * H]
            at_pre = at_pre + _dot(pooled_g, ws[g])
        return jax.nn.relu(at_pre)

    def lstm(et, at, ht, ct):
        g = _dot(et, wie_m) + _dot(at, wia_m) + _dot(ht, whh_m) + bg_r
        ii = jax.nn.sigmoid(g[:, 0 * H:1 * H])
        ff = jax.nn.sigmoid(g[:, 1 * H:2 * H])
        gg = jnp.tanh(g[:, 2 * H:3 * H])
        oo = jax.nn.sigmoid(g[:, 3 * H:4 * H])
        c_new = ff * ct + ii * gg
        h_new = oo * jnp.tanh(c_new)
        return h_new, c_new

    ht = ones_col * h0b[...]
    ct = ones_col * c0b[...]

    for t in range(_OBS_LEN):
        pxc = tx[:, t:t + 1]
        pyc = ty[:, t:t + 1]
        at = social(pxc, pyc, ht)
        et = jax.nn.relu(rx[:, t:t + 1] * wpos0 + ry[:, t:t + 1] * wpos1 + bpos_r)
        ht, ct = lstm(et, at, ht, ct)

    pxc = tx[:, _OBS_LEN - 1:_OBS_LEN]
    pyc = ty[:, _OBS_LEN - 1:_OBS_LEN]
    kmf_r = kmf[...]
    for t in range(_PRED_LEN):
        o = _dot(ht, wpred[...]) + bpredr[...]
        out_ref[t] = o * kmf_r
        dx = o[:, 0:1]
        dy = o[:, 1:2]
        pxc = pxc + dx
        pyc = pyc + dy
        at = social(pxc, pyc, ht)
        et = jax.nn.relu(dx * wpos0 + dy * wpos1 + bpos_r)
        ht, ct = lstm(et, at, ht, ct)


def _prep(traj, traj_rel, time_mask, is_predictable_mask, same_scene_mask, h0, c0,
          W_pos, b_pos, W_soc, b_soc, W_ih, W_hh, b_ih, b_hh, W_pred, b_pred):
    f32 = jnp.float32
    tx = traj[:, :, 0].astype(f32)
    ty = traj[:, :, 1].astype(f32)
    rx = traj_rel[:, :, 0].astype(f32)
    ry = traj_rel[:, :, 1].astype(f32)
    sidf = same_scene_mask.astype(f32)
    kmf = (is_predictable_mask == 1).astype(f32)
    h0b = h0.reshape(1, _HIDDEN)
    c0b = c0.reshape(1, _HIDDEN)
    wposT = W_pos.T
    bpos = b_pos.reshape(1, _EMBED)
    ws = [W_soc[:, g * _HIDDEN:(g + 1) * _HIDDEN].T for g in range(_GRID * _GRID)]
    bsoc = b_soc.reshape(1, _EMBED)
    wie = W_ih[:, :_EMBED].T
    wia = W_ih[:, _EMBED:].T
    whh = W_hh.T
    bg = (b_ih + b_hh).reshape(1, 4 * _HIDDEN)
    wpred = W_pred.T
    bpredr = b_pred.reshape(1, 5)
    return (tx, ty, rx, ry, sidf, kmf, h0b, c0b, wposT, bpos,
            ws[0], ws[1], ws[2], ws[3], bsoc, wie, wia, whh, bg, wpred, bpredr)


def kernel(traj, traj_rel, time_mask, is_predictable_mask, same_scene_mask, h0, c0,
           W_pos, b_pos, W_soc, b_soc, W_ih, W_hh, b_ih, b_hh, W_pred, b_pred):
    ops = _prep(traj, traj_rel, time_mask, is_predictable_mask, same_scene_mask,
                h0, c0, W_pos, b_pos, W_soc, b_soc, W_ih, W_hh, b_ih, b_hh,
                W_pred, b_pred)
    out = pl.pallas_call(
        _body,
        out_shape=jax.ShapeDtypeStruct((_PRED_LEN, _N, 5), jnp.float32),
    )(*ops)
    return jnp.transpose(out, (1, 0, 2))


# speedup vs baseline: 47.8802x; 1.1061x over previous
import jax
import jax.numpy as jnp
from jax.experimental import pallas as pl

_GRID = 2
_SPAN = 2.0
_EMBED = 64
_HIDDEN = 128
_PRED_LEN = 12
_N = 256
_OBS_LEN = 8

_PREC = jax.lax.Precision.HIGHEST


def _dot(a, b):
    return jax.lax.dot_general(a, b, (((1,), (0,)), ((), ())),
                               precision=_PREC, preferred_element_type=jnp.float32)


def _dot_fast(a, b):
    return jax.lax.dot_general(a, b, (((1,), (0,)), ((), ())),
                               precision=jax.lax.Precision.DEFAULT,
                               preferred_element_type=jnp.float32)


def _split3(x):
    x1 = x.astype(jnp.bfloat16)
    r1 = x - x1.astype(jnp.float32)
    x2 = r1.astype(jnp.bfloat16)
    x3 = (r1 - x2.astype(jnp.float32)).astype(jnp.bfloat16)
    return jnp.concatenate([x1, x2, x3], axis=1)


def _split2(x):
    x1 = x.astype(jnp.bfloat16)
    x2 = (x - x1.astype(jnp.float32)).astype(jnp.bfloat16)
    return x1, x2


def _dot3(a1, a2, b12, b1, n):
    r = _dot_fast(a1, b12)
    return r[:, :n] + r[:, n:] + _dot_fast(a2, b1)


def _outer_row(ones_col, col):
    return jax.lax.dot_general(ones_col, col, (((1,), (1,)), ((), ())),
                               precision=_PREC, preferred_element_type=jnp.float32)


def _body(tx, ty, txT, tyT, rx, ry, sidf, kmf, h0b, c0b, wposT, bpos,
          ws0, ws1, ws2, ws3, bsoc, wie12, wie1, wia12, wia1, whh12, whh1,
          bg, wpred, bpredr, out_ref):
    N, H = _N, _HIDDEN
    ones_col = jnp.ones((N, 1), jnp.float32)

    sid_col = sidf[...]
    sid_row_full = _outer_row(ones_col, sid_col)
    same = sid_row_full == sid_col
    ri = jax.lax.broadcasted_iota(jnp.int32, (N, N), 0)
    ci = jax.lax.broadcasted_iota(jnp.int32, (N, N), 1)
    base_static = same & (ri != ci)

    wpos0 = wposT[0:1, :]
    wpos1 = wposT[1:2, :]
    bpos_r = bpos[...]
    bsoc_r = bsoc[...]
    bg_r = bg[...]
    wie12_m, wie1_m = wie12[...], wie1[...]
    wia12_m, wia1_m = wia12[...], wia1[...]
    whh12_m, whh1_m = whh12[...], whh1[...]
    ws = (ws0[...], ws1[...], ws2[...], ws3[...])

    lim = _SPAN / 2.0 - 0.01

    def social(pxr, pyr, pxc, pyc, hs):
        relx = pxr - pxc
        rely = pyr - pyc
        within = (relx < lim) & (relx > -lim) & (rely < lim) & (rely > -lim)
        base = within & base_static
        gx = jnp.floor(relx + _SPAN / 2.0)
        gy = jnp.floor(rely + _SPAN / 2.0)
        gid = gx * float(_GRID) + gy
        at_pre = bsoc_r
        for g in range(_GRID * _GRID):
            mg = (base & (gid == float(g))).astype(jnp.bfloat16)
            parts = _dot_fast(mg, hs)
            pooled_g = parts[:, 0:H] + parts[:, H:2 * H] + parts[:, 2 * H:3 * H]
            at_pre = at_pre + _dot(pooled_g, ws[g])
        return jax.nn.relu(at_pre)

    def lstm(et, at, hs, ct):
        e1, e2 = _split2(et)
        a1, a2 = _split2(at)
        h1 = hs[:, 0:H]
        h2 = hs[:, H:2 * H]
        g = (_dot3(e1, e2, wie12_m, wie1_m, 4 * H)
             + _dot3(a1, a2, wia12_m, wia1_m, 4 * H)
             + _dot3(h1, h2, whh12_m, whh1_m, 4 * H) + bg_r)
        ii = jax.nn.sigmoid(g[:, 0 * H:1 * H])
        ff = jax.nn.sigmoid(g[:, 1 * H:2 * H])
        gg = jnp.tanh(g[:, 2 * H:3 * H])
        oo = jax.nn.sigmoid(g[:, 3 * H:4 * H])
        c_new = ff * ct + ii * gg
        h_new = oo * jnp.tanh(c_new)
        return h_new, c_new

    ones_bf = jnp.ones((N, 1), jnp.bfloat16)

    def row_bcast(col):
        c1 = col.astype(jnp.bfloat16)
        r1 = col - c1.astype(jnp.float32)
        c2 = r1.astype(jnp.bfloat16)
        c3 = (r1 - c2.astype(jnp.float32)).astype(jnp.bfloat16)
        dn = (((1,), (1,)), ((), ()))
        out = jax.lax.dot_general(ones_bf, c1, dn, precision=jax.lax.Precision.DEFAULT,
                                  preferred_element_type=jnp.float32)
        out += jax.lax.dot_general(ones_bf, c2, dn, precision=jax.lax.Precision.DEFAULT,
                                   preferred_element_type=jnp.float32)
        out += jax.lax.dot_general(ones_bf, c3, dn, precision=jax.lax.Precision.DEFAULT,
                                   preferred_element_type=jnp.float32)
        return out

    ht = ones_col * h0b[...]
    ct = ones_col * c0b[...]

    for t in range(_OBS_LEN):
        hs = _split3(ht)
        pxc = tx[:, t:t + 1]
        pyc = ty[:, t:t + 1]
        at = social(txT[t:t + 1, :], tyT[t:t + 1, :], pxc, pyc, hs)
        et = jax.nn.relu(rx[:, t:t + 1] * wpos0 + ry[:, t:t + 1] * wpos1 + bpos_r)
        ht, ct = lstm(et, at, hs, ct)

    pxc = tx[:, _OBS_LEN - 1:_OBS_LEN]
    pyc = ty[:, _OBS_LEN - 1:_OBS_LEN]
    kmf_r = kmf[...]
    for t in range(_PRED_LEN):
        o = _dot(ht, wpred[...]) + bpredr[...]
        out_ref[t] = o * kmf_r
        dx = o[:, 0:1]
        dy = o[:, 1:2]
        pxc = pxc + dx
        pyc = pyc + dy
        hs = _split3(ht)
        at = social(row_bcast(pxc), row_bcast(pyc), pxc, pyc, hs)
        et = jax.nn.relu(dx * wpos0 + dy * wpos1 + bpos_r)
        ht, ct = lstm(et, at, hs, ct)


def _prep(traj, traj_rel, time_mask, is_predictable_mask, same_scene_mask, h0, c0,
          W_pos, b_pos, W_soc, b_soc, W_ih, W_hh, b_ih, b_hh, W_pred, b_pred):
    f32 = jnp.float32
    tx = traj[:, :, 0].astype(f32)
    ty = traj[:, :, 1].astype(f32)
    rx = traj_rel[:, :, 0].astype(f32)
    ry = traj_rel[:, :, 1].astype(f32)
    sidf = same_scene_mask.astype(f32)
    kmf = (is_predictable_mask == 1).astype(f32)
    h0b = h0.reshape(1, _HIDDEN)
    c0b = c0.reshape(1, _HIDDEN)
    wposT = W_pos.T
    bpos = b_pos.reshape(1, _EMBED)
    ws = [W_soc[:, g * _HIDDEN:(g + 1) * _HIDDEN].T for g in range(_GRID * _GRID)]
    bsoc = b_soc.reshape(1, _EMBED)
    def w12(w):
        w1 = w.astype(jnp.bfloat16)
        w2 = (w - w1.astype(f32)).astype(jnp.bfloat16)
        return jnp.concatenate([w1, w2], axis=1), w1

    wie12, wie1 = w12(W_ih[:, :_EMBED].T)
    wia12, wia1 = w12(W_ih[:, _EMBED:].T)
    whh12, whh1 = w12(W_hh.T)
    bg = (b_ih + b_hh).reshape(1, 4 * _HIDDEN)
    wpred = W_pred.T
    bpredr = b_pred.reshape(1, 5)
    return (tx, ty, tx.T, ty.T, rx, ry, sidf, kmf, h0b, c0b, wposT, bpos,
            ws[0], ws[1], ws[2], ws[3], bsoc, wie12, wie1, wia12, wia1,
            whh12, whh1, bg, wpred, bpredr)


def kernel(traj, traj_rel, time_mask, is_predictable_mask, same_scene_mask, h0, c0,
           W_pos, b_pos, W_soc, b_soc, W_ih, W_hh, b_ih, b_hh, W_pred, b_pred):
    ops = _prep(traj, traj_rel, time_mask, is_predictable_mask, same_scene_mask,
                h0, c0, W_pos, b_pos, W_soc, b_soc, W_ih, W_hh, b_ih, b_hh,
                W_pred, b_pred)
    out = pl.pallas_call(
        _body,
        out_shape=jax.ShapeDtypeStruct((_PRED_LEN, _N, 5), jnp.float32),
    )(*ops)
    return jnp.transpose(out, (1, 0, 2))


# speedup vs baseline: 59.0306x; 1.2329x over previous
import jax
import jax.numpy as jnp
from jax.experimental import pallas as pl

_GRID = 2
_SPAN = 2.0
_EMBED = 64
_HIDDEN = 128
_PRED_LEN = 12
_N = 256
_OBS_LEN = 8

_PREC = jax.lax.Precision.HIGHEST


def _dot(a, b):
    return jax.lax.dot_general(a, b, (((1,), (0,)), ((), ())),
                               precision=_PREC, preferred_element_type=jnp.float32)


def _dot_fast(a, b):
    return jax.lax.dot_general(a, b, (((1,), (0,)), ((), ())),
                               precision=jax.lax.Precision.DEFAULT,
                               preferred_element_type=jnp.float32)


def _split3(x):
    x1 = x.astype(jnp.bfloat16)
    r1 = x - x1.astype(jnp.float32)
    x2 = r1.astype(jnp.bfloat16)
    x3 = (r1 - x2.astype(jnp.float32)).astype(jnp.bfloat16)
    return jnp.concatenate([x1, x2, x3], axis=1)


def _split2(x):
    x1 = x.astype(jnp.bfloat16)
    x2 = (x - x1.astype(jnp.float32)).astype(jnp.bfloat16)
    return x1, x2


def _split2cat(x):
    x1, x2 = _split2(x)
    return jnp.concatenate([x1, x2], axis=1)


def _dot3(a1, a2, b12, b1, n):
    r = _dot_fast(a1, b12)
    return r[:, :n] + r[:, n:] + _dot_fast(a2, b1)


def _outer_row(ones_col, col):
    return jax.lax.dot_general(ones_col, col, (((1,), (1,)), ((), ())),
                               precision=_PREC, preferred_element_type=jnp.float32)


def _body(tx, ty, txT, tyT, rx, ry, sidf, kmf, h0b, c0b, wposT, bpos,
          ws12_0, ws1_0, ws12_1, ws1_1, ws12_2, ws1_2, ws12_3, ws1_3,
          bsoc, wie12, wie1, wia12, wia1, whh12, whh1,
          bg, wpred, bpredr, out_ref):
    N, H = _N, _HIDDEN
    ones_col = jnp.ones((N, 1), jnp.float32)

    sid_col = sidf[...]
    sid_row_full = _outer_row(ones_col, sid_col)
    same = sid_row_full == sid_col
    ri = jax.lax.broadcasted_iota(jnp.int32, (N, N), 0)
    ci = jax.lax.broadcasted_iota(jnp.int32, (N, N), 1)
    base_static = same & (ri != ci)

    wpos0 = wposT[0:1, :]
    wpos1 = wposT[1:2, :]
    bpos_r = bpos[...]
    bsoc_r = bsoc[...]
    bg_r = bg[...]
    wie12_m, wie1_m = wie12[...], wie1[...]
    wia12_m, wia1_m = wia12[...], wia1[...]
    whh12_m, whh1_m = whh12[...], whh1[...]
    ws12 = (ws12_0[...], ws12_1[...], ws12_2[...], ws12_3[...])
    ws1 = (ws1_0[...], ws1_1[...], ws1_2[...], ws1_3[...])

    lim = _SPAN / 2.0 - 0.01

    def social(pxr, pyr, pxc, pyc, hs):
        relx = pxr - pxc
        rely = pyr - pyc
        within = (relx < lim) & (relx > -lim) & (rely < lim) & (rely > -lim)
        base = within & base_static
        gx = jnp.floor(relx + _SPAN / 2.0)
        gy = jnp.floor(rely + _SPAN / 2.0)
        gid = gx * float(_GRID) + gy
        at_pre = bsoc_r
        for g in range(_GRID * _GRID):
            mg = (base & (gid == float(g))).astype(jnp.bfloat16)
            parts = _dot_fast(mg, hs)
            pooled_g = parts[:, 0:H] + parts[:, H:2 * H]
            p1, p2 = _split2(pooled_g)
            at_pre = at_pre + _dot3(p1, p2, ws12[g], ws1[g], H)
        return jax.nn.relu(at_pre)

    def lstm(et, at, hs, ct):
        e1, e2 = _split2(et)
        a1, a2 = _split2(at)
        h1 = hs[:, 0:H]
        h2 = hs[:, H:2 * H]
        g = (_dot3(e1, e2, wie12_m, wie1_m, 4 * H)
             + _dot3(a1, a2, wia12_m, wia1_m, 4 * H)
             + _dot3(h1, h2, whh12_m, whh1_m, 4 * H) + bg_r)
        ii = jax.nn.sigmoid(g[:, 0 * H:1 * H])
        ff = jax.nn.sigmoid(g[:, 1 * H:2 * H])
        gg = jnp.tanh(g[:, 2 * H:3 * H])
        oo = jax.nn.sigmoid(g[:, 3 * H:4 * H])
        c_new = ff * ct + ii * gg
        h_new = oo * jnp.tanh(c_new)
        return h_new, c_new

    ones_bf = jnp.ones((N, 1), jnp.bfloat16)

    def row_bcast(col):
        c1 = col.astype(jnp.bfloat16)
        r1 = col - c1.astype(jnp.float32)
        c2 = r1.astype(jnp.bfloat16)
        c3 = (r1 - c2.astype(jnp.float32)).astype(jnp.bfloat16)
        dn = (((1,), (1,)), ((), ()))
        out = jax.lax.dot_general(ones_bf, c1, dn, precision=jax.lax.Precision.DEFAULT,
                                  preferred_element_type=jnp.float32)
        out += jax.lax.dot_general(ones_bf, c2, dn, precision=jax.lax.Precision.DEFAULT,
                                   preferred_element_type=jnp.float32)
        out += jax.lax.dot_general(ones_bf, c3, dn, precision=jax.lax.Precision.DEFAULT,
                                   preferred_element_type=jnp.float32)
        return out

    ht = ones_col * h0b[...]
    ct = ones_col * c0b[...]

    for t in range(_OBS_LEN):
        hs = _split2cat(ht)
        pxc = tx[:, t:t + 1]
        pyc = ty[:, t:t + 1]
        at = social(txT[t:t + 1, :], tyT[t:t + 1, :], pxc, pyc, hs)
        et = jax.nn.relu(rx[:, t:t + 1] * wpos0 + ry[:, t:t + 1] * wpos1 + bpos_r)
        ht, ct = lstm(et, at, hs, ct)

    pxc = tx[:, _OBS_LEN - 1:_OBS_LEN]
    pyc = ty[:, _OBS_LEN - 1:_OBS_LEN]
    kmf_r = kmf[...]
    for t in range(_PRED_LEN):
        o = _dot(ht, wpred[...]) + bpredr[...]
        out_ref[t] = o * kmf_r
        dx = o[:, 0:1]
        dy = o[:, 1:2]
        pxc = pxc + dx
        pyc = pyc + dy
        hs = _split2cat(ht)
        at = social(row_bcast(pxc), row_bcast(pyc), pxc, pyc, hs)
        et = jax.nn.relu(dx * wpos0 + dy * wpos1 + bpos_r)
        ht, ct = lstm(et, at, hs, ct)


def _prep(traj, traj_rel, time_mask, is_predictable_mask, same_scene_mask, h0, c0,
          W_pos, b_pos, W_soc, b_soc, W_ih, W_hh, b_ih, b_hh, W_pred, b_pred):
    f32 = jnp.float32
    tx = traj[:, :, 0].astype(f32)
    ty = traj[:, :, 1].astype(f32)
    rx = traj_rel[:, :, 0].astype(f32)
    ry = traj_rel[:, :, 1].astype(f32)
    sidf = same_scene_mask.astype(f32)
    kmf = (is_predictable_mask == 1).astype(f32)
    h0b = h0.reshape(1, _HIDDEN)
    c0b = c0.reshape(1, _HIDDEN)
    wposT = W_pos.T
    bpos = b_pos.reshape(1, _EMBED)
    def w12(w):
        w1 = w.astype(jnp.bfloat16)
        w2 = (w - w1.astype(f32)).astype(jnp.bfloat16)
        return jnp.concatenate([w1, w2], axis=1), w1

    pad = _HIDDEN - _EMBED
    ws_pairs = [w12(jnp.pad(W_soc[:, g * _HIDDEN:(g + 1) * _HIDDEN].T,
                            ((0, 0), (0, pad))))
                for g in range(_GRID * _GRID)]
    bsoc = jnp.pad(b_soc, (0, pad)).reshape(1, _HIDDEN)
    wie12, wie1 = w12(W_ih[:, :_EMBED].T)
    wia12, wia1 = w12(jnp.pad(W_ih[:, _EMBED:].T, ((0, pad), (0, 0))))
    whh12, whh1 = w12(W_hh.T)
    bg = (b_ih + b_hh).reshape(1, 4 * _HIDDEN)
    wpred = W_pred.T
    bpredr = b_pred.reshape(1, 5)
    return (tx, ty, tx.T, ty.T, rx, ry, sidf, kmf, h0b, c0b, wposT, bpos,
            ws_pairs[0][0], ws_pairs[0][1], ws_pairs[1][0], ws_pairs[1][1],
            ws_pairs[2][0], ws_pairs[2][1], ws_pairs[3][0], ws_pairs[3][1],
            bsoc, wie12, wie1, wia12, wia1,
            whh12, whh1, bg, wpred, bpredr)


def kernel(traj, traj_rel, time_mask, is_predictable_mask, same_scene_mask, h0, c0,
           W_pos, b_pos, W_soc, b_soc, W_ih, W_hh, b_ih, b_hh, W_pred, b_pred):
    ops = _prep(traj, traj_rel, time_mask, is_predictable_mask, same_scene_mask,
                h0, c0, W_pos, b_pos, W_soc, b_soc, W_ih, W_hh, b_ih, b_hh,
                W_pred, b_pred)
    out = pl.pallas_call(
        _body,
        out_shape=jax.ShapeDtypeStruct((_PRED_LEN, _N, 5), jnp.float32),
    )(*ops)
    return jnp.transpose(out, (1, 0, 2))


# speedup vs baseline: 59.9026x; 1.0148x over previous
import jax
import jax.numpy as jnp
from jax.experimental import pallas as pl

_GRID = 2
_SPAN = 2.0
_EMBED = 64
_HIDDEN = 128
_PRED_LEN = 12
_N = 256
_OBS_LEN = 8

_PREC = jax.lax.Precision.HIGHEST


def _dot(a, b):
    return jax.lax.dot_general(a, b, (((1,), (0,)), ((), ())),
                               precision=_PREC, preferred_element_type=jnp.float32)


def _dot_fast(a, b):
    return jax.lax.dot_general(a, b, (((1,), (0,)), ((), ())),
                               precision=jax.lax.Precision.DEFAULT,
                               preferred_element_type=jnp.float32)


def _split3(x):
    x1 = x.astype(jnp.bfloat16)
    r1 = x - x1.astype(jnp.float32)
    x2 = r1.astype(jnp.bfloat16)
    x3 = (r1 - x2.astype(jnp.float32)).astype(jnp.bfloat16)
    return jnp.concatenate([x1, x2, x3], axis=1)


def _split2(x):
    x1 = x.astype(jnp.bfloat16)
    x2 = (x - x1.astype(jnp.float32)).astype(jnp.bfloat16)
    return x1, x2


def _split2cat(x):
    x1, x2 = _split2(x)
    return jnp.concatenate([x1, x2], axis=1)


def _dot3(a1, a2, b12, b1, n):
    r = _dot_fast(a1, b12)
    return r[:, :n] + r[:, n:] + _dot_fast(a2, b1)


def _outer_row(ones_col, col):
    return jax.lax.dot_general(ones_col, col, (((1,), (1,)), ((), ())),
                               precision=_PREC, preferred_element_type=jnp.float32)


def _body(tx, ty, txT, tyT, rx, ry, sidf, kmf, h0b, c0b, wposT, bpos,
          ws12_0, ws1_0, ws12_1, ws1_1, ws12_2, ws1_2, ws12_3, ws1_3,
          bsoc, wie12, wie1, wia12, wia1, whh12, whh1,
          bg, wpred, bpredr, out_ref):
    N, H = _N, _HIDDEN
    ones_col = jnp.ones((N, 1), jnp.float32)

    sid_col = sidf[...]
    sid_row_full = _outer_row(ones_col, sid_col)
    same = sid_row_full == sid_col
    ri = jax.lax.broadcasted_iota(jnp.int32, (N, N), 0)
    ci = jax.lax.broadcasted_iota(jnp.int32, (N, N), 1)
    base_static = same & (ri != ci)

    wpos0 = wposT[0:1, :]
    wpos1 = wposT[1:2, :]
    bpos_r = bpos[...]
    bsoc_r = bsoc[...]
    bg_r = bg[...]
    wie12_m, wie1_m = wie12[...], wie1[...]
    wia12_m, wia1_m = wia12[...], wia1[...]
    whh12_m, whh1_m = whh12[...], whh1[...]
    ws12 = (ws12_0[...], ws12_1[...], ws12_2[...], ws12_3[...])
    ws1 = (ws1_0[...], ws1_1[...], ws1_2[...], ws1_3[...])

    lim = _SPAN / 2.0 - 0.01

    def social(pxr, pyr, pxc, pyc, h1, h2):
        relx = pxr - pxc
        rely = pyr - pyc
        within = (relx < lim) & (relx > -lim) & (rely < lim) & (rely > -lim)
        base = within & base_static
        gx = jnp.floor(relx + _SPAN / 2.0)
        gy = jnp.floor(rely + _SPAN / 2.0)
        gid = gx * float(_GRID) + gy
        at_pre = bsoc_r
        for g in range(_GRID * _GRID):
            mg = (base & (gid == float(g))).astype(jnp.bfloat16)
            pooled_g = _dot_fast(mg, h1) + _dot_fast(mg, h2)
            p1, p2 = _split2(pooled_g)
            at_pre = at_pre + _dot3(p1, p2, ws12[g], ws1[g], H)
        return jax.nn.relu(at_pre)

    def lstm(et, at, h1, h2, ct):
        e1, e2 = _split2(et)
        a1, a2 = _split2(at)
        g = (_dot3(e1, e2, wie12_m, wie1_m, 4 * H)
             + _dot3(a1, a2, wia12_m, wia1_m, 4 * H)
             + _dot3(h1, h2, whh12_m, whh1_m, 4 * H) + bg_r)
        ii = jax.nn.sigmoid(g[:, 0 * H:1 * H])
        ff = jax.nn.sigmoid(g[:, 1 * H:2 * H])
        gg = jnp.tanh(g[:, 2 * H:3 * H])
        oo = jax.nn.sigmoid(g[:, 3 * H:4 * H])
        c_new = ff * ct + ii * gg
        h_new = oo * jnp.tanh(c_new)
        return h_new, c_new

    ones_bf = jnp.ones((N, 1), jnp.bfloat16)

    def row_bcast(col):
        c1 = col.astype(jnp.bfloat16)
        r1 = col - c1.astype(jnp.float32)
        c2 = r1.astype(jnp.bfloat16)
        c3 = (r1 - c2.astype(jnp.float32)).astype(jnp.bfloat16)
        dn = (((1,), (1,)), ((), ()))
        out = jax.lax.dot_general(ones_bf, c1, dn, precision=jax.lax.Precision.DEFAULT,
                                  preferred_element_type=jnp.float32)
        out += jax.lax.dot_general(ones_bf, c2, dn, precision=jax.lax.Precision.DEFAULT,
                                   preferred_element_type=jnp.float32)
        out += jax.lax.dot_general(ones_bf, c3, dn, precision=jax.lax.Precision.DEFAULT,
                                   preferred_element_type=jnp.float32)
        return out

    ht = ones_col * h0b[...]
    ct = ones_col * c0b[...]

    for t in range(_OBS_LEN):
        h1, h2 = _split2(ht)
        pxc = tx[:, t:t + 1]
        pyc = ty[:, t:t + 1]
        at = social(txT[t:t + 1, :], tyT[t:t + 1, :], pxc, pyc, h1, h2)
        et = jax.nn.relu(rx[:, t:t + 1] * wpos0 + ry[:, t:t + 1] * wpos1 + bpos_r)
        ht, ct = lstm(et, at, h1, h2, ct)

    pxc = tx[:, _OBS_LEN - 1:_OBS_LEN]
    pyc = ty[:, _OBS_LEN - 1:_OBS_LEN]
    kmf_r = kmf[...]
    for t in range(_PRED_LEN):
        o = _dot(ht, wpred[...]) + bpredr[...]
        out_ref[t] = o * kmf_r
        dx = o[:, 0:1]
        dy = o[:, 1:2]
        pxc = pxc + dx
        pyc = pyc + dy
        h1, h2 = _split2(ht)
        at = social(row_bcast(pxc), row_bcast(pyc), pxc, pyc, h1, h2)
        et = jax.nn.relu(dx * wpos0 + dy * wpos1 + bpos_r)
        ht, ct = lstm(et, at, h1, h2, ct)


def _prep(traj, traj_rel, time_mask, is_predictable_mask, same_scene_mask, h0, c0,
          W_pos, b_pos, W_soc, b_soc, W_ih, W_hh, b_ih, b_hh, W_pred, b_pred):
    f32 = jnp.float32
    tx = traj[:, :, 0].astype(f32)
    ty = traj[:, :, 1].astype(f32)
    rx = traj_rel[:, :, 0].astype(f32)
    ry = traj_rel[:, :, 1].astype(f32)
    sidf = same_scene_mask.astype(f32)
    kmf = (is_predictable_mask == 1).astype(f32)
    h0b = h0.reshape(1, _HIDDEN)
    c0b = c0.reshape(1, _HIDDEN)
    wposT = W_pos.T
    bpos = b_pos.reshape(1, _EMBED)
    def w12(w):
        w1 = w.astype(jnp.bfloat16)
        w2 = (w - w1.astype(f32)).astype(jnp.bfloat16)
        return jnp.concatenate([w1, w2], axis=1), w1

    pad = _HIDDEN - _EMBED
    ws_pairs = [w12(jnp.pad(W_soc[:, g * _HIDDEN:(g + 1) * _HIDDEN].T,
                            ((0, 0), (0, pad))))
                for g in range(_GRID * _GRID)]
    bsoc = jnp.pad(b_soc, (0, pad)).reshape(1, _HIDDEN)
    wie12, wie1 = w12(W_ih[:, :_EMBED].T)
    wia12, wia1 = w12(jnp.pad(W_ih[:, _EMBED:].T, ((0, pad), (0, 0))))
    whh12, whh1 = w12(W_hh.T)
    bg = (b_ih + b_hh).reshape(1, 4 * _HIDDEN)
    wpred = W_pred.T
    bpredr = b_pred.reshape(1, 5)
    return (tx, ty, tx.T, ty.T, rx, ry, sidf, kmf, h0b, c0b, wposT, bpos,
            ws_pairs[0][0], ws_pairs[0][1], ws_pairs[1][0], ws_pairs[1][1],
            ws_pairs[2][0], ws_pairs[2][1], ws_pairs[3][0], ws_pairs[3][1],
            bsoc, wie12, wie1, wia12, wia1,
            whh12, whh1, bg, wpred, bpredr)


def kernel(traj, traj_rel, time_mask, is_predictable_mask, same_scene_mask, h0, c0,
           W_pos, b_pos, W_soc, b_soc, W_ih, W_hh, b_ih, b_hh, W_pred, b_pred):
    ops = _prep(traj, traj_rel, time_mask, is_predictable_mask, same_scene_mask,
                h0, c0, W_pos, b_pos, W_soc, b_soc, W_ih, W_hh, b_ih, b_hh,
                W_pred, b_pred)
    out = pl.pallas_call(
        _body,
        out_shape=jax.ShapeDtypeStruct((_PRED_LEN, _N, 5), jnp.float32),
    )(*ops)
    return jnp.transpose(out, (1, 0, 2))


# speedup vs baseline: 80.8049x; 1.3489x over previous
import jax
import jax.numpy as jnp
from jax.experimental import pallas as pl

_GRID = 2
_SPAN = 2.0
_EMBED = 64
_HIDDEN = 128
_PRED_LEN = 12
_N = 256
_OBS_LEN = 8

_PREC = jax.lax.Precision.HIGHEST


def _dot(a, b):
    return jax.lax.dot_general(a, b, (((1,), (0,)), ((), ())),
                               precision=_PREC, preferred_element_type=jnp.float32)


def _dot_fast(a, b):
    return jax.lax.dot_general(a, b, (((1,), (0,)), ((), ())),
                               precision=jax.lax.Precision.DEFAULT,
                               preferred_element_type=jnp.float32)


def _split2(x):
    x1 = x.astype(jnp.bfloat16)
    x2 = (x - x1.astype(jnp.float32)).astype(jnp.bfloat16)
    return x1, x2


def _outer_row(ones_col, col):
    return jax.lax.dot_general(ones_col, col, (((1,), (1,)), ((), ())),
                               precision=_PREC, preferred_element_type=jnp.float32)


def _body(tx, ty, txT, tyT, rx, ry, sidf, kmf, h0b, c0b, wposT, bpos,
          wproj, bsoc, wg, bg, wpred, bpredr, out_ref):
    N, H = _N, _HIDDEN
    ones_col = jnp.ones((N, 1), jnp.float32)

    sid_col = sidf[...]
    sid_row_full = _outer_row(ones_col, sid_col)
    same = sid_row_full == sid_col
    ri = jax.lax.broadcasted_iota(jnp.int32, (N, N), 0)
    ci = jax.lax.broadcasted_iota(jnp.int32, (N, N), 1)
    base_static = same & (ri != ci)

    wpos0 = wposT[0:1, :]
    wpos1 = wposT[1:2, :]
    bpos_r = bpos[...]
    bsoc_r = bsoc[...]
    bg_r = bg[...]
    wproj_m = wproj[...]
    wg_m = wg[...]

    lim = _SPAN / 2.0 - 0.01

    def social(pxr, pyr, pxc, pyc, h1, h2):
        relx = pxr - pxc
        rely = pyr - pyc
        within = (relx < lim) & (relx > -lim) & (rely < lim) & (rely > -lim)
        base = within & base_static
        gx = jnp.floor(relx + _SPAN / 2.0)
        gy = jnp.floor(rely + _SPAN / 2.0)
        gid = gx * float(_GRID) + gy
        ps = []
        for g in range(_GRID * _GRID):
            mg = (base & (gid == float(g))).astype(jnp.bfloat16)
            pooled_g = _dot_fast(mg, h1) + _dot_fast(mg, h2)
            p1, p2 = _split2(pooled_g)
            ps += [p1, p1, p2]
        pcat = jnp.concatenate(ps, axis=1)
        return jax.nn.relu(_dot_fast(pcat, wproj_m) + bsoc_r)

    def lstm(et, at, h1, h2, ct):
        e1, e2 = _split2(et)
        a1, a2 = _split2(at)
        xcat = jnp.concatenate([e1, e1, e2, a1, a1, a2, h1, h1, h2], axis=1)
        g = _dot_fast(xcat, wg_m) + bg_r
        ii = jax.nn.sigmoid(g[:, 0 * H:1 * H])
        ff = jax.nn.sigmoid(g[:, 1 * H:2 * H])
        gg = jnp.tanh(g[:, 2 * H:3 * H])
        oo = jax.nn.sigmoid(g[:, 3 * H:4 * H])
        c_new = ff * ct + ii * gg
        h_new = oo * jnp.tanh(c_new)
        return h_new, c_new

    ones3_bf = jnp.ones((N, 3), jnp.bfloat16)

    def row_bcast(col):
        c1 = col.astype(jnp.bfloat16)
        r1 = col - c1.astype(jnp.float32)
        c2 = r1.astype(jnp.bfloat16)
        c3 = (r1 - c2.astype(jnp.float32)).astype(jnp.bfloat16)
        ccat = jnp.concatenate([c1, c2, c3], axis=1)
        return jax.lax.dot_general(ones3_bf, ccat, (((1,), (1,)), ((), ())),
                                   precision=jax.lax.Precision.DEFAULT,
                                   preferred_element_type=jnp.float32)

    ht = ones_col * h0b[...]
    ct = ones_col * c0b[...]

    for t in range(_OBS_LEN):
        h1, h2 = _split2(ht)
        pxc = tx[:, t:t + 1]
        pyc = ty[:, t:t + 1]
        at = social(txT[t:t + 1, :], tyT[t:t + 1, :], pxc, pyc, h1, h2)
        et = jax.nn.relu(rx[:, t:t + 1] * wpos0 + ry[:, t:t + 1] * wpos1 + bpos_r)
        ht, ct = lstm(et, at, h1, h2, ct)

    pxc = tx[:, _OBS_LEN - 1:_OBS_LEN]
    pyc = ty[:, _OBS_LEN - 1:_OBS_LEN]
    kmf_r = kmf[...]
    for t in range(_PRED_LEN):
        o = _dot(ht, wpred[...]) + bpredr[...]
        out_ref[t] = o * kmf_r
        dx = o[:, 0:1]
        dy = o[:, 1:2]
        pxc = pxc + dx
        pyc = pyc + dy
        h1, h2 = _split2(ht)
        at = social(row_bcast(pxc), row_bcast(pyc), pxc, pyc, h1, h2)
        et = jax.nn.relu(dx * wpos0 + dy * wpos1 + bpos_r)
        ht, ct = lstm(et, at, h1, h2, ct)


def _prep(traj, traj_rel, time_mask, is_predictable_mask, same_scene_mask, h0, c0,
          W_pos, b_pos, W_soc, b_soc, W_ih, W_hh, b_ih, b_hh, W_pred, b_pred):
    f32 = jnp.float32
    tx = traj[:, :, 0].astype(f32)
    ty = traj[:, :, 1].astype(f32)
    rx = traj_rel[:, :, 0].astype(f32)
    ry = traj_rel[:, :, 1].astype(f32)
    sidf = same_scene_mask.astype(f32)
    kmf = (is_predictable_mask == 1).astype(f32)
    h0b = h0.reshape(1, _HIDDEN)
    c0b = c0.reshape(1, _HIDDEN)
    pad = _HIDDEN - _EMBED
    wposT = jnp.pad(W_pos.T, ((0, 0), (0, pad)))
    bpos = jnp.pad(b_pos, (0, pad)).reshape(1, _HIDDEN)

    def wsplit(w):
        w1 = w.astype(jnp.bfloat16)
        w2 = (w - w1.astype(f32)).astype(jnp.bfloat16)
        return w1, w2

    proj_rows = []
    for g in range(_GRID * _GRID):
        wsg = jnp.pad(W_soc[:, g * _HIDDEN:(g + 1) * _HIDDEN].T, ((0, 0), (0, pad)))
        w1, w2 = wsplit(wsg)
        proj_rows += [w1, w2, w1]
    wproj = jnp.concatenate(proj_rows, axis=0)
    bsoc = jnp.pad(b_soc, (0, pad)).reshape(1, _HIDDEN)

    wie1, wie2 = wsplit(jnp.pad(W_ih[:, :_EMBED].T, ((0, pad), (0, 0))))
    wia1, wia2 = wsplit(jnp.pad(W_ih[:, _EMBED:].T, ((0, pad), (0, 0))))
    whh1, whh2 = wsplit(W_hh.T)
    wg = jnp.concatenate([wie1, wie2, wie1, wia1, wia2, wia1,
                          whh1, whh2, whh1], axis=0)
    bg = (b_ih + b_hh).reshape(1, 4 * _HIDDEN)
    wpred = W_pred.T
    bpredr = b_pred.reshape(1, 5)
    return (tx, ty, tx.T, ty.T, rx, ry, sidf, kmf, h0b, c0b, wposT, bpos,
            wproj, bsoc, wg, bg, wpred, bpredr)


def kernel(traj, traj_rel, time_mask, is_predictable_mask, same_scene_mask, h0, c0,
           W_pos, b_pos, W_soc, b_soc, W_ih, W_hh, b_ih, b_hh, W_pred, b_pred):
    ops = _prep(traj, traj_rel, time_mask, is_predictable_mask, same_scene_mask,
                h0, c0, W_pos, b_pos, W_soc, b_soc, W_ih, W_hh, b_ih, b_hh,
                W_pred, b_pred)
    out = pl.pallas_call(
        _body,
        out_shape=jax.ShapeDtypeStruct((_PRED_LEN, _N, 5), jnp.float32),
    )(*ops)
    return jnp.transpose(out, (1, 0, 2))


# speedup vs baseline: 86.2447x; 1.0673x over previous
import jax
import jax.numpy as jnp
from jax.experimental import pallas as pl

_GRID = 2
_SPAN = 2.0
_EMBED = 64
_HIDDEN = 128
_PRED_LEN = 12
_N = 256
_OBS_LEN = 8

_PREC = jax.lax.Precision.HIGHEST


def _dot(a, b):
    return jax.lax.dot_general(a, b, (((1,), (0,)), ((), ())),
                               precision=_PREC, preferred_element_type=jnp.float32)


def _dot_fast(a, b):
    return jax.lax.dot_general(a, b, (((1,), (0,)), ((), ())),
                               precision=jax.lax.Precision.DEFAULT,
                               preferred_element_type=jnp.float32)


def _split2(x):
    x1 = x.astype(jnp.bfloat16)
    x2 = (x - x1.astype(jnp.float32)).astype(jnp.bfloat16)
    return x1, x2


def _outer_row(ones_col, col):
    return jax.lax.dot_general(ones_col, col, (((1,), (1,)), ((), ())),
                               precision=_PREC, preferred_element_type=jnp.float32)


def _body(tx, ty, txT, tyT, rx, ry, sidf, kmf, h0b, c0b, wposT, bpos,
          wproj, bsoc, wg, bg, wpred, bpredr, out_ref):
    N, H = _N, _HIDDEN
    ones_col = jnp.ones((N, 1), jnp.float32)

    sid_col = sidf[...]
    sid_row_full = _outer_row(ones_col, sid_col)
    same = sid_row_full == sid_col
    ri = jax.lax.broadcasted_iota(jnp.int32, (N, N), 0)
    ci = jax.lax.broadcasted_iota(jnp.int32, (N, N), 1)
    base_static = same & (ri != ci)

    wpos0 = wposT[0:1, :]
    wpos1 = wposT[1:2, :]
    bpos_r = bpos[...]
    bsoc_r = bsoc[...]
    bg_r = bg[...]
    wproj_m = wproj[...]
    wg_m = wg[...]

    lim = _SPAN / 2.0 - 0.01

    def social(pxr, pyr, pxc, pyc, h1, h2):
        relx = pxr - pxc
        rely = pyr - pyc
        base = (jnp.abs(relx) < lim) & (jnp.abs(rely) < lim) & base_static
        gx = jnp.floor(relx + _SPAN / 2.0)
        gy = jnp.floor(rely + _SPAN / 2.0)
        gid = gx * float(_GRID) + gy
        ps = []
        for g in range(_GRID * _GRID):
            mg = (base & (gid == float(g))).astype(jnp.bfloat16)
            pooled_g = _dot_fast(mg, h1) + _dot_fast(mg, h2)
            p1, p2 = _split2(pooled_g)
            ps += [p1, p1, p2]
        pcat = jnp.concatenate(ps, axis=1)
        return jax.nn.relu(_dot_fast(pcat, wproj_m) + bsoc_r)

    def lstm(et, at, h1, h2, ct):
        e1, e2 = _split2(et)
        a1, a2 = _split2(at)
        xcat = jnp.concatenate([e1, e1, e2, a1, a1, a2, h1, h1, h2], axis=1)
        g = _dot_fast(xcat, wg_m) + bg_r
        ii = jax.nn.sigmoid(g[:, 0 * H:1 * H])
        ff = jax.nn.sigmoid(g[:, 1 * H:2 * H])
        gg = jnp.tanh(g[:, 2 * H:3 * H])
        oo = jax.nn.sigmoid(g[:, 3 * H:4 * H])
        c_new = ff * ct + ii * gg
        h_new = oo * jnp.tanh(c_new)
        return h_new, c_new

    ones3_bf = jnp.ones((N, 3), jnp.bfloat16)

    def row_bcast(col):
        c1 = col.astype(jnp.bfloat16)
        r1 = col - c1.astype(jnp.float32)
        c2 = r1.astype(jnp.bfloat16)
        c3 = (r1 - c2.astype(jnp.float32)).astype(jnp.bfloat16)
        ccat = jnp.concatenate([c1, c2, c3], axis=1)
        return jax.lax.dot_general(ones3_bf, ccat, (((1,), (1,)), ((), ())),
                                   precision=jax.lax.Precision.DEFAULT,
                                   preferred_element_type=jnp.float32)

    ht = ones_col * h0b[...]
    ct = ones_col * c0b[...]

    for t in range(_OBS_LEN):
        h1, h2 = _split2(ht)
        pxc = tx[:, t:t + 1]
        pyc = ty[:, t:t + 1]
        at = social(txT[t:t + 1, :], tyT[t:t + 1, :], pxc, pyc, h1, h2)
        et = jax.nn.relu(rx[:, t:t + 1] * wpos0 + ry[:, t:t + 1] * wpos1 + bpos_r)
        ht, ct = lstm(et, at, h1, h2, ct)

    pxc = tx[:, _OBS_LEN - 1:_OBS_LEN]
    pyc = ty[:, _OBS_LEN - 1:_OBS_LEN]
    kmf_r = kmf[...]
    wpred_m = wpred[...]
    for t in range(_PRED_LEN):
        h1, h2 = _split2(ht)
        o = _dot_fast(jnp.concatenate([h1, h1, h2], axis=1), wpred_m) + bpredr[...]
        out_ref[t] = o * kmf_r
        dx = o[:, 0:1]
        dy = o[:, 1:2]
        pxc = pxc + dx
        pyc = pyc + dy
        at = social(row_bcast(pxc), row_bcast(pyc), pxc, pyc, h1, h2)
        et = jax.nn.relu(dx * wpos0 + dy * wpos1 + bpos_r)
        ht, ct = lstm(et, at, h1, h2, ct)


def _prep(traj, traj_rel, time_mask, is_predictable_mask, same_scene_mask, h0, c0,
          W_pos, b_pos, W_soc, b_soc, W_ih, W_hh, b_ih, b_hh, W_pred, b_pred):
    f32 = jnp.float32
    tx = traj[:, :, 0].astype(f32)
    ty = traj[:, :, 1].astype(f32)
    rx = traj_rel[:, :, 0].astype(f32)
    ry = traj_rel[:, :, 1].astype(f32)
    sidf = same_scene_mask.astype(f32)
    kmf = (is_predictable_mask == 1).astype(f32)
    h0b = h0.reshape(1, _HIDDEN)
    c0b = c0.reshape(1, _HIDDEN)
    pad = _HIDDEN - _EMBED
    wposT = jnp.pad(W_pos.T, ((0, 0), (0, pad)))
    bpos = jnp.pad(b_pos, (0, pad)).reshape(1, _HIDDEN)

    def wsplit(w):
        w1 = w.astype(jnp.bfloat16)
        w2 = (w - w1.astype(f32)).astype(jnp.bfloat16)
        return w1, w2

    proj_rows = []
    for g in range(_GRID * _GRID):
        wsg = jnp.pad(W_soc[:, g * _HIDDEN:(g + 1) * _HIDDEN].T, ((0, 0), (0, pad)))
        w1, w2 = wsplit(wsg)
        proj_rows += [w1, w2, w1]
    wproj = jnp.concatenate(proj_rows, axis=0)
    bsoc = jnp.pad(b_soc, (0, pad)).reshape(1, _HIDDEN)

    wie1, wie2 = wsplit(jnp.pad(W_ih[:, :_EMBED].T, ((0, pad), (0, 0))))
    wia1, wia2 = wsplit(jnp.pad(W_ih[:, _EMBED:].T, ((0, pad), (0, 0))))
    whh1, whh2 = wsplit(W_hh.T)
    wg = jnp.concatenate([wie1, wie2, wie1, wia1, wia2, wia1,
                          whh1, whh2, whh1], axis=0)
    bg = (b_ih + b_hh).reshape(1, 4 * _HIDDEN)
    wp1, wp2 = wsplit(W_pred.T)
    wpred = jnp.concatenate([wp1, wp2, wp1], axis=0)
    bpredr = b_pred.reshape(1, 5)
    return (tx, ty, tx.T, ty.T, rx, ry, sidf, kmf, h0b, c0b, wposT, bpos,
            wproj, bsoc, wg, bg, wpred, bpredr)


def kernel(traj, traj_rel, time_mask, is_predictable_mask, same_scene_mask, h0, c0,
           W_pos, b_pos, W_soc, b_soc, W_ih, W_hh, b_ih, b_hh, W_pred, b_pred):
    ops = _prep(traj, traj_rel, time_mask, is_predictable_mask, same_scene_mask,
                h0, c0, W_pos, b_pos, W_soc, b_soc, W_ih, W_hh, b_ih, b_hh,
                W_pred, b_pred)
    out = pl.pallas_call(
        _body,
        out_shape=jax.ShapeDtypeStruct((_PRED_LEN, _N, 5), jnp.float32),
    )(*ops)
    return jnp.transpose(out, (1, 0, 2))
